# baseline (speedup 1.0000x reference)
import functools
import math

import jax
import jax.numpy as jnp
from jax import lax
from jax.experimental import pallas as pl
from jax.experimental.pallas import tpu as pltpu
from jax.experimental.pallas import tpu_sc as plsc

D_MODEL = 32
PAD_D = 128
SCALE = math.sqrt(D_MODEL)

NC = 2
NS = 16
NW = NC * NS
LANES = 16

NBUF = 2


def _make_gather(S0: int, S1: int):
    assert S0 % NW == 0
    s0_per_w = S0 // NW

    mesh = plsc.VectorSubcoreMesh(core_axis_name="c", subcore_axis_name="s")

    @functools.partial(
        pl.kernel,
        mesh=mesh,
        out_type=jax.ShapeDtypeStruct((S0, S1, D_MODEL), jnp.float32),
        scratch_types=[
            pltpu.VMEM((NBUF, 1, S1), jnp.int32),
            pltpu.VMEM((NBUF, S1, PAD_D), jnp.float32),
            pltpu.VMEM((NBUF, S1, D_MODEL), jnp.float32),
            [pltpu.SemaphoreType.DMA] * NBUF,
            [pltpu.SemaphoreType.DMA] * NBUF,
        ],
    )
    def gather_kernel(table_hbm, idx_hbm, out_hbm, idx_v, rows_v, packed_v,
                      gsems, osems):
        wid = lax.axis_index("s") * NC + lax.axis_index("c")
        row_base = wid * s0_per_w

        def gather_copy(b):
            return pltpu.make_async_copy(
                table_hbm.at[idx_v.at[b, 0]], rows_v.at[b], gsems[b]
            )

        for b in range(NBUF):
            pltpu.sync_copy(idx_hbm.at[pl.ds(row_base + b, 1)], idx_v.at[b])
            gather_copy(b).start()

        def outer(o, carry):
            for b in range(NBUF):
                g = o * NBUF + b
                row = row_base + g
                gather_copy(b).wait()

                def scale_body(r, c):
                    lo = rows_v[b, r, pl.ds(0, LANES)]
                    packed_v[b, r, pl.ds(0, LANES)] = lo * SCALE
                    hi = rows_v[b, r, pl.ds(LANES, LANES)]
                    packed_v[b, r, pl.ds(LANES, LANES)] = hi * SCALE
                    return c

                lax.fori_loop(0, S1, scale_body, 0, unroll=8)

                out_copy = pltpu.make_async_copy(
                    packed_v.at[b], out_hbm.at[row], osems[b]
                )
                out_copy.start()

                not_last = o < s0_per_w // NBUF - 1

                @pl.when(not_last)
                def _prefetch_idx():
                    pltpu.sync_copy(
                        idx_hbm.at[pl.ds(row + NBUF, 1)], idx_v.at[b]
                    )

                out_copy.wait()

                @pl.when(not_last)
                def _launch_gather():
                    gather_copy(b).start()

            return carry

        lax.fori_loop(0, s0_per_w // NBUF, outer, 0)

    return gather_kernel


def kernel(x, table):
    S0, S1 = x.shape
    idx = x.astype(jnp.int32)
    table_pad = jnp.pad(table, ((0, 0), (0, PAD_D - D_MODEL)))
    gather = _make_gather(S0, S1)
    return gather(table_pad, idx)
